# Initial kernel scaffold; baseline (speedup 1.0000x reference)
import functools

import jax
import jax.numpy as jnp
from jax import lax
from jax.experimental import pallas as pl
from jax.experimental.pallas import tpu as pltpu
from jax.experimental.pallas import tpu_sc as plsc

B = 16384
CONT = 13
NF = 26
V = 100000
D = 16

NC = 2
NS = 16
NW = NC * NS
ROWS_W = B // NW
CHUNK = 128
NCHUNK = ROWS_W // CHUNK

OUT_W = CONT + NF * D


def _sc_kernel(cate_t_hbm, cont_hbm, tables_hbm, out_hbm,
               idx_v, em_v, cont_v, sem):
    wid = lax.axis_index("s") * NC + lax.axis_index("c")
    base = wid * ROWS_W

    def chunk_body(c, carry):
        row0 = base + c * CHUNK
        pltpu.sync_copy(cate_t_hbm.at[:, pl.ds(row0, CHUNK)], idx_v)
        pltpu.sync_copy(cont_hbm.at[pl.ds(row0, CHUNK)], cont_v)
        descs = []
        for f in range(NF):
            descs.append(pltpu.async_copy(
                tables_hbm.at[f, idx_v.at[f]], em_v.at[f], sem))
        for d in descs:
            d.wait()
        pltpu.sync_copy(cont_v, out_hbm.at[pl.ds(row0, CHUNK), pl.ds(0, CONT)])
        for f in range(NF):
            pltpu.sync_copy(
                em_v.at[f],
                out_hbm.at[pl.ds(row0, CHUNK), pl.ds(CONT + f * D, D)])
        return carry

    lax.fori_loop(0, NCHUNK, chunk_body, 0)


def kernel(x_cont, x_cate, tables):
    cate_t = x_cate.T
    mesh = plsc.VectorSubcoreMesh(core_axis_name="c", subcore_axis_name="s")
    run = functools.partial(
        pl.kernel,
        mesh=mesh,
        out_type=jax.ShapeDtypeStruct((B, OUT_W), jnp.float32),
        scratch_types=[
            pltpu.VMEM((NF, CHUNK), jnp.int32),
            pltpu.VMEM((NF, CHUNK, D), jnp.float32),
            pltpu.VMEM((CHUNK, CONT), jnp.float32),
            pltpu.SemaphoreType.DMA,
        ],
    )(_sc_kernel)
    return run(cate_t, x_cont, tables)


# speedup vs baseline: 1.1310x; 1.1310x over previous
import functools

import jax
import jax.numpy as jnp
from jax import lax
from jax.experimental import pallas as pl
from jax.experimental.pallas import tpu as pltpu
from jax.experimental.pallas import tpu_sc as plsc

B = 16384
CONT = 13
NF = 26
V = 100000
D = 16

NC = 2
NS = 16
NW = NC * NS
ROWS_W = B // NW
G = 128
NG_W = ROWS_W * NF // G
CHUNK_G = NF
NCHUNK = NG_W // CHUNK_G
PERIOD = 13


def _sc_kernel(cate_hbm, offs_hbm, table_hbm, out_hbm,
               idx_v, off_v, em_v, sem):
    wid = lax.axis_index("s") * NC + lax.axis_index("c")

    pltpu.sync_copy(cate_hbm.at[pl.ds(wid * NG_W, NG_W)], idx_v)
    pltpu.sync_copy(offs_hbm, off_v)

    def fix_body(g, carry):
        p = lax.rem(g, PERIOD)
        for k in range(G // 16):
            s = pl.ds(k * 16, 16)
            idx_v[g, s] = idx_v[g, s] + off_v[p, s]
        return carry

    lax.fori_loop(0, NG_W, fix_body, 0)

    def chunk_body(c, carry):
        descs = []
        for f in range(CHUNK_G):
            descs.append(pltpu.async_copy(
                table_hbm.at[idx_v.at[c * CHUNK_G + f]],
                em_v.at[pl.ds(f * G, G)], sem))
        for dsc in descs:
            dsc.wait()
        pltpu.sync_copy(
            em_v, out_hbm.at[pl.ds((wid * NCHUNK + c) * CHUNK_G * G,
                                   CHUNK_G * G)])
        return carry

    lax.fori_loop(0, NCHUNK, chunk_body, 0)


@jax.jit
def kernel(x_cont, x_cate, tables):
    cate2d = x_cate.reshape(B * NF // G, G)
    table_flat = tables.reshape(NF * V, D)
    offs = ((jnp.arange(PERIOD * G, dtype=jnp.int32) % NF) * V
            ).reshape(PERIOD, G)
    mesh = plsc.VectorSubcoreMesh(core_axis_name="c", subcore_axis_name="s")
    run = functools.partial(
        pl.kernel,
        mesh=mesh,
        compiler_params=pltpu.CompilerParams(use_tc_tiling_on_sc=False),
        out_type=jax.ShapeDtypeStruct((B * NF, D), jnp.float32),
        scratch_types=[
            pltpu.VMEM((NG_W, G), jnp.int32),
            pltpu.VMEM((PERIOD, G), jnp.int32),
            pltpu.VMEM((CHUNK_G * G, D), jnp.float32),
            pltpu.SemaphoreType.DMA,
        ],
    )(_sc_kernel)
    em = run(cate2d, offs, table_flat)
    return jnp.concatenate([x_cont, em.reshape(B, NF * D)], axis=1)
